# baseline (speedup 1.0000x reference)
import functools

import jax
import jax.numpy as jnp
from jax import lax
from jax.experimental import pallas as pl
from jax.experimental.pallas import tpu as pltpu
from jax.experimental.pallas import tpu_sc as plsc


def kernel(inputs, table, gamma, beta):
    B, S, H = inputs.shape
    info = plsc.get_sparse_core_info()
    nc, ns = info.num_cores, info.num_subcores
    nw = nc * ns
    rows_per_w = S // nw
    chunk = 32
    n_chunks = rows_per_w // chunk

    mesh = plsc.VectorSubcoreMesh(core_axis_name="c", subcore_axis_name="s")

    @functools.partial(
        pl.kernel,
        mesh=mesh,
        out_type=jax.ShapeDtypeStruct((B, S, H), jnp.float32),
        scratch_types=[
            pltpu.VMEM((2, chunk, H), jnp.float32),
            pltpu.SemaphoreType.DMA,
            pltpu.SemaphoreType.DMA,
            pltpu.SemaphoreType.DMA,
            pltpu.SemaphoreType.DMA,
        ],
    )
    def broadcast_rows(table_hbm, out_hbm, bufs, isem0, isem1, osem0, osem1):
        isems = (isem0, isem1)
        osems = (osem0, osem1)
        wid = lax.axis_index("s") * nc + lax.axis_index("c")
        base = wid * rows_per_w

        def in_slice(c):
            return table_hbm.at[pl.ds(base + c * chunk, chunk)]

        def out_slice(b, c):
            return out_hbm.at[b, pl.ds(base + c * chunk, chunk)]

        pltpu.async_copy(in_slice(0), bufs.at[0], isems[0])
        for c in range(n_chunks):
            i = c % 2
            pltpu.make_async_copy(in_slice(c), bufs.at[i], isems[i]).wait()
            for b in range(B):
                pltpu.async_copy(bufs.at[i], out_slice(b, c), osems[i])
            if c + 1 < n_chunks:
                j = (c + 1) % 2
                if c >= 1:
                    for b in range(B):
                        pltpu.make_async_copy(
                            bufs.at[j], out_slice(b, c - 1), osems[j]
                        ).wait()
                pltpu.async_copy(in_slice(c + 1), bufs.at[j], isems[j])
        for c in (n_chunks - 2, n_chunks - 1):
            i = c % 2
            for b in range(B):
                pltpu.make_async_copy(bufs.at[i], out_slice(b, c), osems[i]).wait()

    return broadcast_rows(table)


# speedup vs baseline: 1.0532x; 1.0532x over previous
import functools

import jax
import jax.numpy as jnp
from jax import lax
from jax.experimental import pallas as pl
from jax.experimental.pallas import tpu as pltpu
from jax.experimental.pallas import tpu_sc as plsc


def kernel(inputs, table, gamma, beta):
    B, S, H = inputs.shape
    info = plsc.get_sparse_core_info()
    nc, ns = info.num_cores, info.num_subcores
    nw = nc * ns
    rows_per_w = S // nw
    chunk = 64
    n_chunks = rows_per_w // chunk

    mesh = plsc.VectorSubcoreMesh(core_axis_name="c", subcore_axis_name="s")

    @functools.partial(
        pl.kernel,
        mesh=mesh,
        out_type=jax.ShapeDtypeStruct((B, S, H), jnp.float32),
        scratch_types=[pltpu.VMEM((chunk, H), jnp.float32)],
    )
    def broadcast_rows(table_hbm, out_hbm, buf):
        wid = lax.axis_index("s") * nc + lax.axis_index("c")
        base = wid * rows_per_w
        for c in range(n_chunks):
            r0 = base + c * chunk
            pltpu.sync_copy(table_hbm.at[pl.ds(r0, chunk)], buf)
            for b in range(B):
                pltpu.sync_copy(buf, out_hbm.at[b, pl.ds(r0, chunk)])

    return broadcast_rows(table)
